# baseline (speedup 1.0000x reference)
import functools

import jax
import jax.numpy as jnp
from jax import lax
from jax.experimental import pallas as pl
from jax.experimental.pallas import tpu as pltpu
from jax.experimental.pallas import tpu_sc as plsc

_REORDER_RATIO = 0.5
_MIN_W = 2
_MAX_W = 5
_LANES = 16


def _row_uniforms(key, batch):
    base = key
    keys = jax.vmap(lambda i: jax.random.fold_in(base, i))(
        jnp.arange(batch, dtype=jnp.int32))

    def draws(k):
        ku, kw, ks, kp = jax.random.split(k, 4)
        return (jax.random.uniform(ku), jax.random.uniform(kw),
                jax.random.uniform(ks), jax.random.uniform(kp, (_MAX_W,)))

    uu, uw, us, r = jax.vmap(draws)(keys)
    return uu, uw, us, r


def _build_aux(item_seq_len, batch):
    uu, uw, us, r = _row_uniforms(jax.random.key(1), batch)
    slen = item_seq_len.astype(jnp.int32)
    flag = ((slen > _MIN_W) & (uu <= _REORDER_RATIO)).astype(jnp.int32)
    col = lax.broadcasted_iota(jnp.int32, (batch, 16), 1)
    aux_f = jnp.zeros((batch, 16), jnp.float32)
    for j in range(_MAX_W):
        aux_f = jnp.where(col == j, r[:, j, None], aux_f)
    aux_f = jnp.where(col == 5, uw[:, None], aux_f)
    aux_f = jnp.where(col == 6, us[:, None], aux_f)
    aux_i = jnp.where(col == 0, flag[:, None],
                      jnp.where(col == 1, slen[:, None], 0))
    return aux_f, aux_i


_GROUP = 128


def _sc_body(seq_hbm, auxf_hbm, auxi_hbm, out_hbm, seq_v, auxf_v, auxi_v,
             pk_v, seq_v2, auxf_v2, auxi_v2, in_sem0, in_sem1, out_sem0,
             out_sem1, *, n_rows, length, n_workers):
    rows_per = n_rows // n_workers
    n_groups = rows_per // _GROUP
    wid = lax.axis_index("s") * 2 + lax.axis_index("c")
    base = wid * rows_per

    lanes = lax.iota(jnp.int32, _LANES)
    inf_v = jnp.full((_LANES,), jnp.inf, jnp.float32)

    seq_bufs = (seq_v, seq_v2)
    auxf_bufs = (auxf_v, auxf_v2)
    auxi_bufs = (auxi_v, auxi_v2)
    in_sems = (in_sem0, in_sem1)
    out_sems = (out_sem0, out_sem1)

    def start_in(g):
        gbase = base + g * _GROUP
        b = g % 2
        c1 = pltpu.make_async_copy(seq_hbm.at[pl.ds(gbase, _GROUP)],
                                   seq_bufs[b], in_sems[b])
        c2 = pltpu.make_async_copy(auxf_hbm.at[pl.ds(gbase, _GROUP)],
                                   auxf_bufs[b], in_sems[b])
        c3 = pltpu.make_async_copy(auxi_hbm.at[pl.ds(gbase, _GROUP)],
                                   auxi_bufs[b], in_sems[b])
        c1.start()
        c2.start()
        c3.start()
        return (c1, c2, c3)

    handles_in = {0: start_in(0)}
    handles_out = {}
    for g in range(n_groups):
        b = g % 2
        if g + 1 < n_groups:
            if g - 1 >= 0:
                for h in handles_out[g - 1]:
                    h.wait()
            handles_in[g + 1] = start_in(g + 1)
        for h in handles_in[g]:
            h.wait()
        _sc_group(seq_bufs[b], auxf_bufs[b], auxi_bufs[b], pk_v, lanes,
                  inf_v, length)
        gbase = base + g * _GROUP
        co = pltpu.make_async_copy(seq_bufs[b],
                                   out_hbm.at[pl.ds(gbase, _GROUP)],
                                   out_sems[b])
        co.start()
        handles_out[g] = (co,)
    for g in (n_groups - 2, n_groups - 1):
        if g >= 0:
            for h in handles_out[g]:
                h.wait()


def _sc_group(seq_v, auxf_v, auxi_v, pk_v, lanes, inf_v, length):
    big_v = jnp.full((_LANES,), jnp.int32(1 << 30), jnp.int32)

    def row_fn(r, _):
        auxirow = auxi_v[r, :]
        flag = auxirow[0]

        @pl.when(flag != 0)
        def _process():
            auxrow_f = auxf_v[r, :]
            slen = auxirow[1]
            slen_v = jnp.full((_LANES,), slen, jnp.int32)
            nch = jnp.minimum((slen + (_LANES - 1)) // _LANES, 12)

            def compact(off, posv, carry):
                v = seq_v[r, pl.ds(off, _LANES)]
                m = (v != 0) & (posv < slen_v)
                packed, _ = plsc.sort_key_val(jnp.where(m, posv, big_v),
                                              posv)
                cnt = plsc.all_reduce_population_count(m)
                plsc.store_scatter(pk_v, [carry + lanes], packed,
                                   mask=lanes < cnt)
                return carry + cnt

            def chunk_fn(c, carry):
                off = c * _LANES
                return compact(off, lanes + off, carry)

            nv0 = lax.fori_loop(0, nch, chunk_fn,
                                jnp.zeros((_LANES,), jnp.int32))

            def tail_fn(carry):
                off = length - _LANES
                posv = lanes + off
                v = seq_v[r, pl.ds(off, _LANES)]
                m = (v != 0) & (posv < slen_v) & (posv >= 12 * _LANES)
                packed, _ = plsc.sort_key_val(jnp.where(m, posv, big_v),
                                              posv)
                cnt = plsc.all_reduce_population_count(m)
                plsc.store_scatter(pk_v, [carry + lanes], packed,
                                   mask=lanes < cnt)
                return carry + cnt

            nv = lax.cond(slen > 12 * _LANES, tail_fn, lambda c: c, nv0)

            nv_f = nv.astype(jnp.float32)
            uw_f = jnp.full((_LANES,), auxrow_f[5], jnp.float32)
            us_f = jnp.full((_LANES,), auxrow_f[6], jnp.float32)

            maxp = jnp.minimum(nv_f, float(_MAX_W))
            span = jnp.maximum(maxp - (_MIN_W - 1), 1.0)
            ws = _MIN_W + (uw_f * span).astype(jnp.int32)
            ws = jnp.clip(ws, _MIN_W,
                          jnp.maximum(maxp.astype(jnp.int32), _MIN_W))
            max_start = jnp.maximum(nv_f - ws.astype(jnp.float32) + 1.0, 1.0)
            start = (us_f * max_start).astype(jnp.int32)

            tgt = jnp.clip(start + lanes, 0, length - 1)
            win_pos = plsc.load_gather(pk_v, [tgt])
            win_pos = jnp.clip(win_pos, 0, length - 1)
            r_splat = jnp.full((_LANES,), r, jnp.int32)
            win_items = plsc.load_gather(seq_v, [r_splat, win_pos])

            in_win = lanes < ws
            key = jnp.where(in_win, auxrow_f, inf_v)
            _, shuffled = plsc.sort_key_val(key, win_items)
            do_write = in_win & (nv >= _MIN_W)
            plsc.store_scatter(seq_v, [r_splat, win_pos], shuffled,
                               mask=do_write)

    lax.fori_loop(0, _GROUP, row_fn, None)


def _run_sc(item_seq, aux_f, aux_i):
    batch, length = item_seq.shape
    n_workers = 32
    rows_per = batch // n_workers
    mesh = plsc.VectorSubcoreMesh(core_axis_name="c", subcore_axis_name="s")
    f = pl.kernel(
        functools.partial(_sc_body, n_rows=batch, length=length,
                          n_workers=n_workers),
        mesh=mesh,
        compiler_params=pltpu.CompilerParams(use_tc_tiling_on_sc=False, needs_layout_passes=False),
        out_type=jax.ShapeDtypeStruct((batch, length), jnp.int32),
        scratch_types=[
            pltpu.VMEM((_GROUP, 200), jnp.int32),
            pltpu.VMEM((_GROUP, 16), jnp.float32),
            pltpu.VMEM((_GROUP, 16), jnp.int32),
            pltpu.VMEM((256,), jnp.int32),
            pltpu.VMEM((_GROUP, 200), jnp.int32),
            pltpu.VMEM((_GROUP, 16), jnp.float32),
            pltpu.VMEM((_GROUP, 16), jnp.int32),
            pltpu.SemaphoreType.DMA,
            pltpu.SemaphoreType.DMA,
            pltpu.SemaphoreType.DMA,
            pltpu.SemaphoreType.DMA,
        ],
    )
    return f(item_seq, aux_f, aux_i)


def kernel(item_seq, item_seq_len):
    batch, _ = item_seq.shape
    aux_f, aux_i = _build_aux(item_seq_len, batch)
    out = _run_sc(item_seq, aux_f, aux_i)
    return out, item_seq_len


# speedup vs baseline: 1.0540x; 1.0540x over previous
import functools

import jax
import jax.numpy as jnp
from jax import lax
from jax.experimental import pallas as pl
from jax.experimental.pallas import tpu as pltpu
from jax.experimental.pallas import tpu_sc as plsc

_REORDER_RATIO = 0.5
_MIN_W = 2
_MAX_W = 5
_LANES = 16


def _row_uniforms(key, batch):
    base = key
    keys = jax.vmap(lambda i: jax.random.fold_in(base, i))(
        jnp.arange(batch, dtype=jnp.int32))

    def draws(k):
        ku, kw, ks, kp = jax.random.split(k, 4)
        return (jax.random.uniform(ku), jax.random.uniform(kw),
                jax.random.uniform(ks), jax.random.uniform(kp, (_MAX_W,)))

    uu, uw, us, r = jax.vmap(draws)(keys)
    return uu, uw, us, r


def _build_aux(item_seq_len, batch):
    uu, uw, us, r = _row_uniforms(jax.random.key(1), batch)
    slen = item_seq_len.astype(jnp.int32)
    flag = ((slen > _MIN_W) & (uu <= _REORDER_RATIO)).astype(jnp.int32)
    zf = jnp.zeros((batch, 9), jnp.float32)
    aux_f = jnp.concatenate([r, uw[:, None], us[:, None], zf], axis=1)
    zi = jnp.zeros((batch, 14), jnp.int32)
    aux_i = jnp.concatenate([flag[:, None], slen[:, None], zi], axis=1)
    return aux_f, aux_i


_GROUP = 128


def _sc_body(seq_hbm, auxf_hbm, auxi_hbm, out_hbm, seq_v, auxf_v, auxi_v,
             pk_v, seq_v2, auxf_v2, auxi_v2, in_sem0, in_sem1, out_sem0,
             out_sem1, *, n_rows, length, n_workers):
    rows_per = n_rows // n_workers
    n_groups = rows_per // _GROUP
    wid = lax.axis_index("s") * 2 + lax.axis_index("c")
    base = wid * rows_per

    lanes = lax.iota(jnp.int32, _LANES)
    inf_v = jnp.full((_LANES,), jnp.inf, jnp.float32)

    seq_bufs = (seq_v, seq_v2)
    auxf_bufs = (auxf_v, auxf_v2)
    auxi_bufs = (auxi_v, auxi_v2)
    in_sems = (in_sem0, in_sem1)
    out_sems = (out_sem0, out_sem1)

    def start_in(g):
        gbase = base + g * _GROUP
        b = g % 2
        c1 = pltpu.make_async_copy(seq_hbm.at[pl.ds(gbase, _GROUP)],
                                   seq_bufs[b], in_sems[b])
        c2 = pltpu.make_async_copy(auxf_hbm.at[pl.ds(gbase, _GROUP)],
                                   auxf_bufs[b], in_sems[b])
        c3 = pltpu.make_async_copy(auxi_hbm.at[pl.ds(gbase, _GROUP)],
                                   auxi_bufs[b], in_sems[b])
        c1.start()
        c2.start()
        c3.start()
        return (c1, c2, c3)

    handles_in = {0: start_in(0)}
    handles_out = {}
    for g in range(n_groups):
        b = g % 2
        if g + 1 < n_groups:
            if g - 1 >= 0:
                for h in handles_out[g - 1]:
                    h.wait()
            handles_in[g + 1] = start_in(g + 1)
        for h in handles_in[g]:
            h.wait()
        _sc_group(seq_bufs[b], auxf_bufs[b], auxi_bufs[b], pk_v, lanes,
                  inf_v, length)
        gbase = base + g * _GROUP
        co = pltpu.make_async_copy(seq_bufs[b],
                                   out_hbm.at[pl.ds(gbase, _GROUP)],
                                   out_sems[b])
        co.start()
        handles_out[g] = (co,)
    for g in (n_groups - 2, n_groups - 1):
        if g >= 0:
            for h in handles_out[g]:
                h.wait()


def _sc_group(seq_v, auxf_v, auxi_v, pk_v, lanes, inf_v, length):
    big_v = jnp.full((_LANES,), jnp.int32(1 << 30), jnp.int32)

    def row_fn(r, _):
        auxirow = auxi_v[r, :]
        flag = auxirow[0]

        @pl.when(flag != 0)
        def _process():
            auxrow_f = auxf_v[r, :]
            slen = auxirow[1]
            slen_v = jnp.full((_LANES,), slen, jnp.int32)
            nch = jnp.minimum((slen + (_LANES - 1)) // _LANES, 12)

            def compact(off, posv, carry):
                v = seq_v[r, pl.ds(off, _LANES)]
                m = (v != 0) & (posv < slen_v)
                packed, _ = plsc.sort_key_val(jnp.where(m, posv, big_v),
                                              posv)
                cnt = plsc.all_reduce_population_count(m)
                plsc.store_scatter(pk_v, [carry + lanes], packed,
                                   mask=lanes < cnt)
                return carry + cnt

            def chunk_fn(c, carry):
                off = c * _LANES
                return compact(off, lanes + off, carry)

            nv0 = lax.fori_loop(0, nch, chunk_fn,
                                jnp.zeros((_LANES,), jnp.int32))

            def tail_fn(carry):
                off = length - _LANES
                posv = lanes + off
                v = seq_v[r, pl.ds(off, _LANES)]
                m = (v != 0) & (posv < slen_v) & (posv >= 12 * _LANES)
                packed, _ = plsc.sort_key_val(jnp.where(m, posv, big_v),
                                              posv)
                cnt = plsc.all_reduce_population_count(m)
                plsc.store_scatter(pk_v, [carry + lanes], packed,
                                   mask=lanes < cnt)
                return carry + cnt

            nv = lax.cond(slen > 12 * _LANES, tail_fn, lambda c: c, nv0)

            nv_f = nv.astype(jnp.float32)
            uw_f = jnp.full((_LANES,), auxrow_f[5], jnp.float32)
            us_f = jnp.full((_LANES,), auxrow_f[6], jnp.float32)

            maxp = jnp.minimum(nv_f, float(_MAX_W))
            span = jnp.maximum(maxp - (_MIN_W - 1), 1.0)
            ws = _MIN_W + (uw_f * span).astype(jnp.int32)
            ws = jnp.clip(ws, _MIN_W,
                          jnp.maximum(maxp.astype(jnp.int32), _MIN_W))
            max_start = jnp.maximum(nv_f - ws.astype(jnp.float32) + 1.0, 1.0)
            start = (us_f * max_start).astype(jnp.int32)

            tgt = jnp.clip(start + lanes, 0, length - 1)
            win_pos = plsc.load_gather(pk_v, [tgt])
            win_pos = jnp.clip(win_pos, 0, length - 1)
            r_splat = jnp.full((_LANES,), r, jnp.int32)
            win_items = plsc.load_gather(seq_v, [r_splat, win_pos])

            in_win = lanes < ws
            key = jnp.where(in_win, auxrow_f, inf_v)
            _, shuffled = plsc.sort_key_val(key, win_items)
            do_write = in_win & (nv >= _MIN_W)
            plsc.store_scatter(seq_v, [r_splat, win_pos], shuffled,
                               mask=do_write)

    lax.fori_loop(0, _GROUP, row_fn, None)


def _run_sc(item_seq, aux_f, aux_i):
    batch, length = item_seq.shape
    n_workers = 32
    rows_per = batch // n_workers
    mesh = plsc.VectorSubcoreMesh(core_axis_name="c", subcore_axis_name="s")
    f = pl.kernel(
        functools.partial(_sc_body, n_rows=batch, length=length,
                          n_workers=n_workers),
        mesh=mesh,
        compiler_params=pltpu.CompilerParams(use_tc_tiling_on_sc=False, needs_layout_passes=False),
        out_type=jax.ShapeDtypeStruct((batch, length), jnp.int32),
        scratch_types=[
            pltpu.VMEM((_GROUP, 200), jnp.int32),
            pltpu.VMEM((_GROUP, 16), jnp.float32),
            pltpu.VMEM((_GROUP, 16), jnp.int32),
            pltpu.VMEM((256,), jnp.int32),
            pltpu.VMEM((_GROUP, 200), jnp.int32),
            pltpu.VMEM((_GROUP, 16), jnp.float32),
            pltpu.VMEM((_GROUP, 16), jnp.int32),
            pltpu.SemaphoreType.DMA,
            pltpu.SemaphoreType.DMA,
            pltpu.SemaphoreType.DMA,
            pltpu.SemaphoreType.DMA,
        ],
    )
    return f(item_seq, aux_f, aux_i)


def kernel(item_seq, item_seq_len):
    batch, _ = item_seq.shape
    aux_f, aux_i = _build_aux(item_seq_len, batch)
    out = _run_sc(item_seq, aux_f, aux_i)
    return out, item_seq_len
